# baseline (speedup 1.0000x reference)
import functools

import jax
import jax.numpy as jnp
from jax.experimental import pallas as pl
from jax.experimental.pallas import tpu as pltpu


def _rvq_body(x_ref, cb_ref, enc_ref, q_ref, hn_ref, split_ref, *, n_q, k, d):
    @pl.when(pl.program_id(0) == 0)
    def _():
        cb = cb_ref[...]
        hn_ref[...] = 0.5 * jnp.sum(cb * cb, axis=2)
        hi = cb.astype(jnp.bfloat16)
        r1 = cb - hi.astype(jnp.float32)
        mid = r1.astype(jnp.bfloat16)
        lo = (r1 - mid.astype(jnp.float32)).astype(jnp.bfloat16)
        split_ref[...] = jnp.concatenate([hi, mid, lo], axis=2)

    xb = x_ref[...]
    bn = xb.shape[0]
    lane_iota = jax.lax.broadcasted_iota(jnp.int32, (bn, k), 1)
    q = jnp.zeros_like(xb)
    idx_cols = []
    dnum_t = (((1,), (1,)), ((), ()))
    dnum = (((1,), (0,)), ((), ()))
    for j in range(n_q):
        r = (xb - q).astype(jnp.bfloat16)
        scores = jax.lax.dot_general(
            r, split_ref[j, :, :d], dnum_t,
            preferred_element_type=jnp.float32)
        scores = scores - hn_ref[j][None, :]
        m = jnp.max(scores, axis=1, keepdims=True)
        idx = jnp.min(jnp.where(scores == m, lane_iota, k), axis=1)
        idx_cols.append(idx)
        onehot = (lane_iota == idx[:, None]).astype(jnp.float32)
        onehot = onehot.astype(jnp.bfloat16)
        g = jax.lax.dot_general(onehot, split_ref[j], dnum,
                                preferred_element_type=jnp.float32)
        q = q + ((g[:, :d] + g[:, d:2 * d]) + g[:, 2 * d:])
    enc_ref[...] = jnp.stack(idx_cols, axis=1)
    q_ref[...] = q


@jax.jit
def kernel(x, codebooks):
    n, d = x.shape
    n_q, k, _ = codebooks.shape
    bn = 2048
    enc, quant = pl.pallas_call(
        functools.partial(_rvq_body, n_q=n_q, k=k, d=d),
        grid=(n // bn,),
        in_specs=[
            pl.BlockSpec((bn, d), lambda i: (i, 0)),
            pl.BlockSpec((n_q, k, d), lambda i: (0, 0, 0)),
        ],
        out_specs=[
            pl.BlockSpec((bn, n_q), lambda i: (i, 0)),
            pl.BlockSpec((bn, d), lambda i: (i, 0)),
        ],
        out_shape=[
            jax.ShapeDtypeStruct((n, n_q), jnp.int32),
            jax.ShapeDtypeStruct((n, d), jnp.float32),
        ],
        scratch_shapes=[
            pltpu.VMEM((n_q, k), jnp.float32),
            pltpu.VMEM((n_q, k, 3 * d), jnp.bfloat16),
        ],
    )(x, codebooks)
    return (enc, quant)


# speedup vs baseline: 1.0370x; 1.0370x over previous
import functools

import jax
import jax.numpy as jnp
from jax.experimental import pallas as pl
from jax.experimental.pallas import tpu as pltpu


def _rvq_body(x_ref, cb_ref, enc_ref, q_ref, hn_ref, split_ref, *, n_q, k, d):
    @pl.when(pl.program_id(0) == 0)
    def _():
        cb = cb_ref[...]
        hn_ref[...] = 0.5 * jnp.sum(cb * cb, axis=2)
        hi = cb.astype(jnp.bfloat16)
        r1 = cb - hi.astype(jnp.float32)
        mid = r1.astype(jnp.bfloat16)
        lo = (r1 - mid.astype(jnp.float32)).astype(jnp.bfloat16)
        split_ref[...] = jnp.concatenate([hi, mid, lo], axis=2)

    xb = x_ref[...]
    bn = xb.shape[0]
    lane_iota = jax.lax.broadcasted_iota(jnp.int32, (bn, k), 1)
    q = jnp.zeros_like(xb)
    idx_cols = []
    dnum_t = (((1,), (1,)), ((), ()))
    dnum = (((1,), (0,)), ((), ()))
    for j in range(n_q):
        r = (xb - q).astype(jnp.bfloat16)
        scores = jax.lax.dot_general(
            r, split_ref[j, :, :d], dnum_t,
            preferred_element_type=jnp.float32)
        scores = scores - hn_ref[j][None, :]
        m = jnp.max(scores, axis=1, keepdims=True)
        idx = jnp.min(jnp.where(scores == m, lane_iota, k), axis=1)
        idx_cols.append(idx)
        onehot = (lane_iota == idx[:, None]).astype(jnp.float32)
        onehot = onehot.astype(jnp.bfloat16)
        g = jax.lax.dot_general(onehot, split_ref[j], dnum,
                                preferred_element_type=jnp.float32)
        q = q + ((g[:, :d] + g[:, d:2 * d]) + g[:, 2 * d:])
    enc_ref[...] = jnp.stack(idx_cols, axis=1)
    q_ref[...] = q


@jax.jit
def kernel(x, codebooks):
    n, d = x.shape
    n_q, k, _ = codebooks.shape
    bn = 512
    enc, quant = pl.pallas_call(
        functools.partial(_rvq_body, n_q=n_q, k=k, d=d),
        grid=(n // bn,),
        in_specs=[
            pl.BlockSpec((bn, d), lambda i: (i, 0)),
            pl.BlockSpec((n_q, k, d), lambda i: (0, 0, 0)),
        ],
        out_specs=[
            pl.BlockSpec((bn, n_q), lambda i: (i, 0)),
            pl.BlockSpec((bn, d), lambda i: (i, 0)),
        ],
        out_shape=[
            jax.ShapeDtypeStruct((n, n_q), jnp.int32),
            jax.ShapeDtypeStruct((n, d), jnp.float32),
        ],
        scratch_shapes=[
            pltpu.VMEM((n_q, k), jnp.float32),
            pltpu.VMEM((n_q, k, 3 * d), jnp.bfloat16),
        ],
    )(x, codebooks)
    return (enc, quant)


# speedup vs baseline: 1.4442x; 1.3927x over previous
import functools

import jax
import jax.numpy as jnp
from jax.experimental import pallas as pl
from jax.experimental.pallas import tpu as pltpu


def _rvq_body(x_ref, cb_ref, enc_ref, q_ref, sc_ref, split_ref, *, n_q, k, d):
    @pl.when(pl.program_id(0) == 0)
    def _():
        cb = cb_ref[...]
        hi = cb.astype(jnp.bfloat16)
        r1 = cb - hi.astype(jnp.float32)
        mid = r1.astype(jnp.bfloat16)
        lo = (r1 - mid.astype(jnp.float32)).astype(jnp.bfloat16)
        split_ref[...] = jnp.concatenate([hi, mid, lo], axis=2)
        nhn = -0.5 * jnp.sum(cb * cb, axis=2, keepdims=True)
        h1 = nhn.astype(jnp.bfloat16)
        s1 = nhn - h1.astype(jnp.float32)
        h2 = s1.astype(jnp.bfloat16)
        h3 = (s1 - h2.astype(jnp.float32)).astype(jnp.bfloat16)
        zpad = jnp.zeros((n_q, k, 5), jnp.bfloat16)
        sc_ref[...] = jnp.concatenate([hi, h1, h2, h3, zpad], axis=2)

    xb = x_ref[...]
    bn = xb.shape[0]
    lane_iota = jax.lax.broadcasted_iota(jnp.int32, (bn, k), 1)
    ones3 = jnp.concatenate(
        [jnp.ones((bn, 3), jnp.bfloat16), jnp.zeros((bn, 5), jnp.bfloat16)],
        axis=1)
    q = jnp.zeros_like(xb)
    idx_cols = []
    dnum_t = (((1,), (1,)), ((), ()))
    dnum = (((1,), (0,)), ((), ()))
    for j in range(n_q):
        r = (xb - q).astype(jnp.bfloat16)
        r_aug = jnp.concatenate([r, ones3], axis=1)
        scores = jax.lax.dot_general(
            r_aug, sc_ref[j], dnum_t,
            preferred_element_type=jnp.float32)
        m = jnp.max(scores, axis=1, keepdims=True)
        eq = scores == m
        idx = jnp.min(jnp.where(eq, lane_iota, k), axis=1)
        idx_cols.append(idx)
        onehot = jnp.where(eq, 1.0, 0.0).astype(jnp.bfloat16)
        g = jax.lax.dot_general(onehot, split_ref[j], dnum,
                                preferred_element_type=jnp.float32)
        q = q + ((g[:, :d] + g[:, d:2 * d]) + g[:, 2 * d:])
    enc_ref[...] = jnp.stack(idx_cols, axis=1)
    q_ref[...] = q


@jax.jit
def kernel(x, codebooks):
    n, d = x.shape
    n_q, k, _ = codebooks.shape
    bn = 1024
    enc, quant = pl.pallas_call(
        functools.partial(_rvq_body, n_q=n_q, k=k, d=d),
        grid=(n // bn,),
        in_specs=[
            pl.BlockSpec((bn, d), lambda i: (i, 0)),
            pl.BlockSpec((n_q, k, d), lambda i: (0, 0, 0)),
        ],
        out_specs=[
            pl.BlockSpec((bn, n_q), lambda i: (i, 0)),
            pl.BlockSpec((bn, d), lambda i: (i, 0)),
        ],
        out_shape=[
            jax.ShapeDtypeStruct((n, n_q), jnp.int32),
            jax.ShapeDtypeStruct((n, d), jnp.float32),
        ],
        scratch_shapes=[
            pltpu.VMEM((n_q, k, d + 8), jnp.bfloat16),
            pltpu.VMEM((n_q, k, 3 * d), jnp.bfloat16),
        ],
    )(x, codebooks)
    return (enc, quant)
